# baseline (device time: 47283 ns/iter reference)
import jax
import jax.numpy as jnp
from jax import lax
from jax.experimental import pallas as pl
from jax.experimental.pallas import tpu as pltpu

N_DEV = 4
B = 2
SQ = 256
SKV = 256
DH = 64
H_LOC = 4
D_OUT = 512


def _body(x_ref, wq_ref, k_ref, v_ref, wo_ref, out_ref,
          comm_ref, send_sems, recv_sems):
    my = lax.axis_index("i")
    left = lax.rem(my + N_DEV - 1, N_DEV)
    right = lax.rem(my + 1, N_DEV)

    barrier_sem = pltpu.get_barrier_semaphore()
    for nbr in (left, right):
        pl.semaphore_signal(barrier_sem, inc=1, device_id=(nbr,),
                            device_id_type=pl.DeviceIdType.MESH)
    pl.semaphore_wait(barrier_sem, 2)

    rows = lax.broadcasted_iota(jnp.int32, (SQ, SKV), 0) // 64
    cols = lax.broadcasted_iota(jnp.int32, (SQ, SKV), 1) // 64
    mask = (rows == cols) | ((cols % 4) == (rows % 4))

    for b in range(B):
        q_b = jnp.dot(x_ref[b], wq_ref[...],
                      preferred_element_type=jnp.float32)
        ctx_parts = []
        for h in range(H_LOC):
            q_h = q_b[:, h * DH:(h + 1) * DH]
            k_h = k_ref[b, h]
            v_h = v_ref[b, h]
            scores = lax.dot_general(
                q_h, k_h, (((1,), (1,)), ((), ())),
                preferred_element_type=jnp.float32) * 0.125
            scores = jnp.where(mask, scores, -1e9)
            m = jnp.max(scores, axis=-1, keepdims=True)
            w = jnp.exp(scores - m)
            w = w / jnp.sum(w, axis=-1, keepdims=True)
            ctx_parts.append(jnp.dot(w, v_h,
                                     preferred_element_type=jnp.float32))
        ctx_b = jnp.concatenate(ctx_parts, axis=1)
        partial_b = jnp.dot(ctx_b, wo_ref[...],
                            preferred_element_type=jnp.float32)
        out_ref[b] = partial_b
        comm_ref[0, b] = partial_b

    for h in range(N_DEV - 1):
        rdma = pltpu.make_async_remote_copy(
            src_ref=comm_ref.at[h],
            dst_ref=comm_ref.at[h + 1],
            send_sem=send_sems.at[h],
            recv_sem=recv_sems.at[h],
            device_id=(right,),
            device_id_type=pl.DeviceIdType.MESH,
        )
        rdma.start()
        rdma.wait()
        out_ref[...] = out_ref[...] + comm_ref[h + 1]


def kernel(x, Wq, K_ext, V_ext, Wo):
    my = lax.axis_index("i")
    k_loc = jnp.moveaxis(
        lax.dynamic_slice_in_dim(K_ext, my * H_LOC, H_LOC, axis=2), 2, 1)
    v_loc = jnp.moveaxis(
        lax.dynamic_slice_in_dim(V_ext, my * H_LOC, H_LOC, axis=2), 2, 1)

    return pl.pallas_call(
        _body,
        out_shape=jax.ShapeDtypeStruct((B, SQ, D_OUT), jnp.float32),
        in_specs=[pl.BlockSpec(memory_space=pltpu.VMEM)] * 5,
        out_specs=pl.BlockSpec(memory_space=pltpu.VMEM),
        scratch_shapes=[
            pltpu.VMEM((N_DEV, B, SQ, D_OUT), jnp.float32),
            pltpu.SemaphoreType.DMA((N_DEV - 1,)),
            pltpu.SemaphoreType.DMA((N_DEV - 1,)),
        ],
        compiler_params=pltpu.CompilerParams(collective_id=0),
    )(x, Wq, k_loc, v_loc, Wo)


# device time: 24712 ns/iter; 1.9134x vs baseline; 1.9134x over previous
import jax
import jax.numpy as jnp
from jax import lax
from jax.experimental import pallas as pl
from jax.experimental.pallas import tpu as pltpu

N_DEV = 4
B = 2
SQ = 256
SKV = 256
DH = 64
H_LOC = 4
D_OUT = 512
QR = 128


def _body(x_ref, wq_ref, k_ref, v_ref, wo_ref, out_ref,
          acc_ref, rbuf_ref, rs_ssem, rs_rsem, ag_ssem, ag_rsem):
    my = lax.axis_index("i")

    barrier_sem = pltpu.get_barrier_semaphore()
    for k in (1, 2, 3):
        pl.semaphore_signal(barrier_sem, inc=1, device_id=(my ^ k,),
                            device_id_type=pl.DeviceIdType.MESH)
    pl.semaphore_wait(barrier_sem, 3)

    rows = lax.broadcasted_iota(jnp.int32, (SQ, SKV), 0) // 64
    cols = lax.broadcasted_iota(jnp.int32, (SQ, SKV), 1) // 64
    mask = (rows == cols) | ((cols % 4) == (rows % 4))

    for b in range(B):
        q_b = jnp.dot(x_ref[b], wq_ref[...],
                      preferred_element_type=jnp.float32)
        ctx_parts = []
        for h in range(H_LOC):
            q_h = q_b[:, h * DH:(h + 1) * DH]
            k_h = k_ref[b, h]
            v_h = v_ref[b, h]
            scores = lax.dot_general(
                q_h, k_h, (((1,), (1,)), ((), ())),
                preferred_element_type=jnp.float32) * 0.125
            scores = jnp.where(mask, scores, -1e9)
            m = jnp.max(scores, axis=-1, keepdims=True)
            w = jnp.exp(scores - m)
            w = w / jnp.sum(w, axis=-1, keepdims=True)
            ctx_parts.append(jnp.dot(w, v_h,
                                     preferred_element_type=jnp.float32))
        ctx_b = jnp.concatenate(ctx_parts, axis=1)
        partial_b = jnp.dot(ctx_b, wo_ref[...],
                            preferred_element_type=jnp.float32)
        acc_ref[2 * b] = partial_b[0:QR]
        acc_ref[2 * b + 1] = partial_b[QR:SQ]

    rs = []
    for k in (1, 2, 3):
        peer = my ^ k
        r = pltpu.make_async_remote_copy(
            src_ref=acc_ref.at[peer],
            dst_ref=rbuf_ref.at[k - 1],
            send_sem=rs_ssem.at[k - 1],
            recv_sem=rs_rsem.at[k - 1],
            device_id=(peer,),
            device_id_type=pl.DeviceIdType.MESH,
        )
        r.start()
        rs.append(r)
    for r in rs:
        r.wait_recv()
    total = rbuf_ref[0] + rbuf_ref[1] + rbuf_ref[2]
    for q in range(N_DEV):
        @pl.when(my == q)
        def _(q=q):
            acc_ref[q] = acc_ref[q] + total
    for r in rs:
        r.wait_send()

    ag = []
    for k in (1, 2, 3):
        peer = my ^ k
        r = pltpu.make_async_remote_copy(
            src_ref=acc_ref.at[my],
            dst_ref=acc_ref.at[my],
            send_sem=ag_ssem.at[k - 1],
            recv_sem=ag_rsem.at[k - 1],
            device_id=(peer,),
            device_id_type=pl.DeviceIdType.MESH,
        )
        r.start()
        ag.append(r)
    for r in ag:
        r.wait_recv()
    for r in ag:
        r.wait_send()

    out_ref[0, 0:QR] = acc_ref[0]
    out_ref[0, QR:SQ] = acc_ref[1]
    out_ref[1, 0:QR] = acc_ref[2]
    out_ref[1, QR:SQ] = acc_ref[3]


def kernel(x, Wq, K_ext, V_ext, Wo):
    my = lax.axis_index("i")
    k_loc = jnp.moveaxis(
        lax.dynamic_slice_in_dim(K_ext, my * H_LOC, H_LOC, axis=2), 2, 1)
    v_loc = jnp.moveaxis(
        lax.dynamic_slice_in_dim(V_ext, my * H_LOC, H_LOC, axis=2), 2, 1)

    return pl.pallas_call(
        _body,
        out_shape=jax.ShapeDtypeStruct((B, SQ, D_OUT), jnp.float32),
        in_specs=[pl.BlockSpec(memory_space=pltpu.VMEM)] * 5,
        out_specs=pl.BlockSpec(memory_space=pltpu.VMEM),
        scratch_shapes=[
            pltpu.VMEM((N_DEV, QR, D_OUT), jnp.float32),
            pltpu.VMEM((3, QR, D_OUT), jnp.float32),
            pltpu.SemaphoreType.DMA((3,)),
            pltpu.SemaphoreType.DMA((3,)),
            pltpu.SemaphoreType.DMA((3,)),
            pltpu.SemaphoreType.DMA((3,)),
        ],
        compiler_params=pltpu.CompilerParams(collective_id=0),
    )(x, Wq, k_loc, v_loc, Wo)


# device time: 19125 ns/iter; 2.4723x vs baseline; 1.2921x over previous
import jax
import jax.numpy as jnp
from jax import lax
from jax.experimental import pallas as pl
from jax.experimental.pallas import tpu as pltpu

N_DEV = 4
B = 2
SQ = 256
SKV = 256
DH = 64
H_LOC = 4
D_OUT = 512
QR = 128


def _body(x_ref, wq_ref, k_ref, v_ref, wo_ref, out_ref,
          acc_ref, sbuf_ref, rbuf_ref, abuf_ref,
          rs_ssem, rs_rsem, ag_ssem, ag_rsem):
    my = lax.axis_index("i")

    barrier_sem = pltpu.get_barrier_semaphore()
    for k in (1, 2, 3):
        pl.semaphore_signal(barrier_sem, inc=1, device_id=(my ^ k,),
                            device_id_type=pl.DeviceIdType.MESH)
    pl.semaphore_wait(barrier_sem, 3)

    rows = lax.broadcasted_iota(jnp.int32, (SQ, SKV), 0) // 64
    cols = lax.broadcasted_iota(jnp.int32, (SQ, SKV), 1) // 64
    mask = (rows == cols) | ((cols % 4) == (rows % 4))

    for b in range(B):
        q_b = jnp.dot(x_ref[b], wq_ref[...],
                      preferred_element_type=jnp.float32)
        ctx_parts = []
        for h in range(H_LOC):
            q_h = q_b[:, h * DH:(h + 1) * DH]
            k_h = k_ref[b, h]
            v_h = v_ref[b, h]
            scores = lax.dot_general(
                q_h, k_h, (((1,), (1,)), ((), ())),
                preferred_element_type=jnp.float32) * 0.125
            scores = jnp.where(mask, scores, -1e9)
            m = jnp.max(scores, axis=-1, keepdims=True)
            w = jnp.exp(scores - m)
            w = w / jnp.sum(w, axis=-1, keepdims=True)
            ctx_parts.append(jnp.dot(w, v_h,
                                     preferred_element_type=jnp.float32))
        ctx_b = jnp.concatenate(ctx_parts, axis=1)
        partial_b = jnp.dot(ctx_b, wo_ref[...],
                            preferred_element_type=jnp.float32)
        acc_ref[2 * b] = partial_b[0:QR]
        acc_ref[2 * b + 1] = partial_b[QR:SQ]
        sbuf_ref[2 * b] = partial_b[0:QR].astype(jnp.bfloat16)
        sbuf_ref[2 * b + 1] = partial_b[QR:SQ].astype(jnp.bfloat16)

    rs = []
    for k in (1, 2, 3):
        peer = my ^ k
        r = pltpu.make_async_remote_copy(
            src_ref=sbuf_ref.at[peer],
            dst_ref=rbuf_ref.at[k - 1],
            send_sem=rs_ssem.at[k - 1],
            recv_sem=rs_rsem.at[k - 1],
            device_id=(peer,),
            device_id_type=pl.DeviceIdType.MESH,
        )
        r.start()
        rs.append(r)
    for r in rs:
        r.wait_recv()
    total = (rbuf_ref[0].astype(jnp.float32)
             + rbuf_ref[1].astype(jnp.float32)
             + rbuf_ref[2].astype(jnp.float32))
    for q in range(N_DEV):
        @pl.when(my == q)
        def _(q=q):
            reduced = acc_ref[q] + total
            acc_ref[q] = reduced
            abuf_ref[q] = reduced.astype(jnp.bfloat16)
    for r in rs:
        r.wait_send()

    ag = []
    for k in (1, 2, 3):
        peer = my ^ k
        r = pltpu.make_async_remote_copy(
            src_ref=abuf_ref.at[my],
            dst_ref=abuf_ref.at[my],
            send_sem=ag_ssem.at[k - 1],
            recv_sem=ag_rsem.at[k - 1],
            device_id=(peer,),
            device_id_type=pl.DeviceIdType.MESH,
        )
        r.start()
        ag.append(r)
    for r in ag:
        r.wait_recv()
    for r in ag:
        r.wait_send()

    for q in range(N_DEV):
        val = jnp.where(my == q, acc_ref[q],
                        abuf_ref[q].astype(jnp.float32))
        out_ref[q // 2, (q % 2) * QR:(q % 2 + 1) * QR] = val


def kernel(x, Wq, K_ext, V_ext, Wo):
    my = lax.axis_index("i")
    k_loc = jnp.moveaxis(
        lax.dynamic_slice_in_dim(K_ext, my * H_LOC, H_LOC, axis=2), 2, 1)
    v_loc = jnp.moveaxis(
        lax.dynamic_slice_in_dim(V_ext, my * H_LOC, H_LOC, axis=2), 2, 1)

    return pl.pallas_call(
        _body,
        out_shape=jax.ShapeDtypeStruct((B, SQ, D_OUT), jnp.float32),
        in_specs=[pl.BlockSpec(memory_space=pltpu.VMEM)] * 5,
        out_specs=pl.BlockSpec(memory_space=pltpu.VMEM),
        scratch_shapes=[
            pltpu.VMEM((N_DEV, QR, D_OUT), jnp.float32),
            pltpu.VMEM((N_DEV, QR, D_OUT), jnp.bfloat16),
            pltpu.VMEM((3, QR, D_OUT), jnp.bfloat16),
            pltpu.VMEM((N_DEV, QR, D_OUT), jnp.bfloat16),
            pltpu.SemaphoreType.DMA((3,)),
            pltpu.SemaphoreType.DMA((3,)),
            pltpu.SemaphoreType.DMA((3,)),
            pltpu.SemaphoreType.DMA((3,)),
        ],
        compiler_params=pltpu.CompilerParams(collective_id=0),
    )(x, Wq, k_loc, v_loc, Wo)
